# baseline (speedup 1.0000x reference)
import functools

import jax
import jax.numpy as jnp
from jax.experimental import pallas as pl
from jax.experimental.pallas import tpu as pltpu

_NBUF = 4


def _gcn_body(nchunks, bm, x_ref, wt_ref, b_ref, p_ref, a_hbm, out_ref,
              xt_ref, abuf, sems):
    xt_ref[...] = (
        jnp.dot(x_ref[...], wt_ref[...], preferred_element_type=jnp.float32)
        + b_ref[...]
    )

    def copy_in(j, slot):
        pltpu.make_async_copy(
            a_hbm.at[pl.ds(j * bm, bm), :], abuf.at[slot], sems.at[slot]
        ).start()

    for j in range(_NBUF):
        copy_in(j, j)

    p = p_ref[0, 0]

    def step(j, _):
        slot = jax.lax.rem(j, _NBUF)
        pltpu.make_async_copy(
            a_hbm.at[pl.ds(j * bm, bm), :], abuf.at[slot], sems.at[slot]
        ).wait()
        acc = jnp.dot(abuf[slot], xt_ref[...], preferred_element_type=jnp.float32)
        out_ref[pl.ds(j * bm, bm), :] = jnp.where(acc >= 0, acc, p * acc)

        @pl.when(j + _NBUF < nchunks)
        def _():
            copy_in(j + _NBUF, slot)

        return _

    jax.lax.fori_loop(0, nchunks, step, None)


def kernel(x, a, W, b, prelu_w):
    n, d_in = x.shape[1], x.shape[2]
    d_out = W.shape[0]
    x2 = x[0]
    wt = W.T
    b2 = b.reshape(1, d_out)
    p2 = prelu_w.reshape(1, 1)

    bm = 200
    nchunks = n // bm

    out = pl.pallas_call(
        functools.partial(_gcn_body, nchunks, bm),
        in_specs=[
            pl.BlockSpec((n, d_in), lambda: (0, 0)),
            pl.BlockSpec((d_in, d_out), lambda: (0, 0)),
            pl.BlockSpec((1, d_out), lambda: (0, 0)),
            pl.BlockSpec((1, 1), lambda: (0, 0)),
            pl.BlockSpec(memory_space=pltpu.HBM),
        ],
        out_specs=pl.BlockSpec((n, d_out), lambda: (0, 0)),
        out_shape=jax.ShapeDtypeStruct((n, d_out), jnp.float32),
        scratch_shapes=[
            pltpu.VMEM((n, d_out), jnp.float32),
            pltpu.VMEM((_NBUF, bm, n), jnp.float32),
            pltpu.SemaphoreType.DMA((_NBUF,)),
        ],
    )(x2, wt, b2, p2, a)
    return out[None]


# speedup vs baseline: 1.0009x; 1.0009x over previous
import functools

import jax
import jax.numpy as jnp
from jax.experimental import pallas as pl
from jax.experimental.pallas import tpu as pltpu

_NBUF = 4


def _gcn_body(nchunks, bm, x_ref, wt_ref, b_ref, p_ref, a_hbm, out_ref,
              xt_ref, abuf, sems):
    xt_ref[...] = (
        jnp.dot(x_ref[...], wt_ref[...], preferred_element_type=jnp.float32)
        + b_ref[...]
    )

    def copy_in(j, slot):
        pltpu.make_async_copy(
            a_hbm.at[pl.ds(j * bm, bm), :], abuf.at[slot], sems.at[slot]
        ).start()

    for j in range(_NBUF):
        copy_in(j, j)

    p = p_ref[0, 0]

    for j in range(nchunks):
        slot = j % _NBUF
        pltpu.make_async_copy(
            a_hbm.at[pl.ds(j * bm, bm), :], abuf.at[slot], sems.at[slot]
        ).wait()
        acc = jnp.dot(abuf[slot], xt_ref[...], preferred_element_type=jnp.float32)
        out_ref[pl.ds(j * bm, bm), :] = jnp.where(acc >= 0, acc, p * acc)
        if j + _NBUF < nchunks:
            copy_in(j + _NBUF, slot)


def kernel(x, a, W, b, prelu_w):
    n, d_in = x.shape[1], x.shape[2]
    d_out = W.shape[0]
    x2 = x[0]
    wt = W.T
    b2 = b.reshape(1, d_out)
    p2 = prelu_w.reshape(1, 1)

    bm = 200
    nchunks = n // bm

    out = pl.pallas_call(
        functools.partial(_gcn_body, nchunks, bm),
        in_specs=[
            pl.BlockSpec((n, d_in), lambda: (0, 0)),
            pl.BlockSpec((d_in, d_out), lambda: (0, 0)),
            pl.BlockSpec((1, d_out), lambda: (0, 0)),
            pl.BlockSpec((1, 1), lambda: (0, 0)),
            pl.BlockSpec(memory_space=pltpu.HBM),
        ],
        out_specs=pl.BlockSpec((n, d_out), lambda: (0, 0)),
        out_shape=jax.ShapeDtypeStruct((n, d_out), jnp.float32),
        scratch_shapes=[
            pltpu.VMEM((n, d_out), jnp.float32),
            pltpu.VMEM((_NBUF, bm, n), jnp.float32),
            pltpu.SemaphoreType.DMA((_NBUF,)),
        ],
    )(x2, wt, b2, p2, a)
    return out[None]
